# baseline (speedup 1.0000x reference)
import functools

import jax
import jax.numpy as jnp
from jax import lax
from jax.experimental import pallas as pl
from jax.experimental.pallas import tpu as pltpu
from jax.experimental.pallas import tpu_sc as plsc

N_SRC = 50000
N_DST = 10000
E = 800000
D_IN = 64
D_OUT = 16
NEG_SLOPE = 0.2

NCORES = 2
NSUB = 16
LANES = 16
EDGE_CHUNK = 2000
NFEAT = 4
NRANGE = 8
RN_CH = 128
RN_PAD = 79 * 128


def _prep_body(x_ref, w_ref, as_ref, ad_ref, hT_ref, asrc_ref, xad_ref):
    w = w_ref[...]
    x1 = x_ref[0:N_DST, :]
    hT = lax.dot_general(w, x1, (((0,), (1,)), ((), ())),
                         preferred_element_type=jnp.float32)
    hT_ref[...] = hT
    asrc_ref[...] = jnp.dot(as_ref[...], hT)
    wd = jnp.dot(w, ad_ref[...])
    xad_ref[...] = jnp.dot(x_ref[...], wd)


_prep = pl.pallas_call(
    _prep_body,
    out_shape=[
        jax.ShapeDtypeStruct((D_OUT, N_DST), jnp.float32),
        jax.ShapeDtypeStruct((N_DST,), jnp.float32),
        jax.ShapeDtypeStruct((N_SRC,), jnp.float32),
    ],
)


def _edge_body(hT, asrc, xadH, rn2H, srcH, dstH, zeros,
               outp, denp,
               asrc_v, adst_v,
               hc0, hc1, hc2, hc3, oc0, oc1, oc2, oc3,
               den_v, rn2_v,
               s_v0, s_v1, d_v0, d_v1,
               sem_s0, sem_s1, sem_d0, sem_d1, sem_g):
    c = lax.axis_index("c")
    s = lax.axis_index("s")
    s_v = (s_v0, s_v1)
    d_v = (d_v0, d_v1)
    sem_s = (sem_s0, sem_s1)
    sem_d = (sem_d0, sem_d1)
    hcol = (hc0, hc1, hc2, hc3)
    ocol = (oc0, oc1, oc2, oc3)

    fpart = s % NFEAT
    r = (c * NSUB + s) // NFEAT
    base = r * (E // NRANGE)
    n_chunks = (E // NRANGE) // EDGE_CHUNK

    def _start(j, b):
        off = base + j * EDGE_CHUNK
        pltpu.async_copy(srcH.at[pl.ds(off, EDGE_CHUNK)], s_v[b], sem_s[b])
        pltpu.async_copy(dstH.at[pl.ds(off, EDGE_CHUNK)], d_v[b], sem_d[b])

    def _wait(j, b):
        off = base + j * EDGE_CHUNK
        pltpu.make_async_copy(
            srcH.at[pl.ds(off, EDGE_CHUNK)], s_v[b], sem_s[b]).wait()
        pltpu.make_async_copy(
            dstH.at[pl.ds(off, EDGE_CHUNK)], d_v[b], sem_d[b]).wait()

    _start(0, 0)
    _start(1, 1)

    pltpu.sync_copy(rn2H, rn2_v)
    for j in range(RN_PAD // RN_CH):
        pltpu.async_copy(xadH.at[rn2_v.at[j]],
                         adst_v.at[pl.ds(j * RN_CH, RN_CH)], sem_g)

    pltpu.sync_copy(asrc, asrc_v)
    for k in range(NFEAT):
        pltpu.sync_copy(hT.at[fpart * NFEAT + k], hcol[k])
        pltpu.sync_copy(zeros, ocol[k])
    pltpu.sync_copy(zeros, den_v)

    for j in range(RN_PAD // RN_CH):
        pltpu.make_async_copy(xadH.at[rn2_v.at[j]],
                              adst_v.at[pl.ds(j * RN_CH, RN_CH)], sem_g).wait()

    def chunk_pair(jo, _):
        for b in range(2):
            j = 2 * jo + b
            _wait(j, b)

            @plsc.parallel_loop(0, EDGE_CHUNK // LANES, unroll=16)
            def _(i):
                sl = pl.ds(i * LANES, LANES)
                si = s_v[b][sl]
                di = d_v[b][sl]
                a = (plsc.load_gather(asrc_v, [si])
                     + plsc.load_gather(adst_v, [di]))
                a = jnp.where(a >= 0.0, a, a * NEG_SLOPE)
                ex = jnp.exp(a)
                plsc.addupdate_scatter(den_v, [di], ex)
                for k in range(NFEAT):
                    hv = plsc.load_gather(hcol[k], [si])
                    plsc.addupdate_scatter(ocol[k], [di], ex * hv)

            @pl.when(j + 2 < n_chunks)
            def _():
                _start(j + 2, b)
        return 0
    lax.fori_loop(0, n_chunks // 2, chunk_pair, 0)

    for k in range(NFEAT):
        pltpu.sync_copy(ocol[k], outp.at[r, fpart * NFEAT + k])

    @pl.when(fpart == 0)
    def _():
        pltpu.sync_copy(den_v, denp.at[r])


_edge = pl.kernel(
    _edge_body,
    out_type=[
        jax.ShapeDtypeStruct((NRANGE, D_OUT, N_DST), jnp.float32),
        jax.ShapeDtypeStruct((NRANGE, N_DST), jnp.float32),
    ],
    mesh=plsc.VectorSubcoreMesh(core_axis_name="c", subcore_axis_name="s"),
    compiler_params=pltpu.CompilerParams(needs_layout_passes=False),
    scratch_types=[
        pltpu.VMEM((N_DST,), jnp.float32),
        pltpu.VMEM((RN_PAD,), jnp.float32),
        pltpu.VMEM((N_DST,), jnp.float32),
        pltpu.VMEM((N_DST,), jnp.float32),
        pltpu.VMEM((N_DST,), jnp.float32),
        pltpu.VMEM((N_DST,), jnp.float32),
        pltpu.VMEM((N_DST,), jnp.float32),
        pltpu.VMEM((N_DST,), jnp.float32),
        pltpu.VMEM((N_DST,), jnp.float32),
        pltpu.VMEM((N_DST,), jnp.float32),
        pltpu.VMEM((N_DST,), jnp.float32),
        pltpu.VMEM((RN_PAD // RN_CH, RN_CH), jnp.int32),
        pltpu.VMEM((EDGE_CHUNK,), jnp.int32),
        pltpu.VMEM((EDGE_CHUNK,), jnp.int32),
        pltpu.VMEM((EDGE_CHUNK,), jnp.int32),
        pltpu.VMEM((EDGE_CHUNK,), jnp.int32),
        pltpu.SemaphoreType.DMA,
        pltpu.SemaphoreType.DMA,
        pltpu.SemaphoreType.DMA,
        pltpu.SemaphoreType.DMA,
        pltpu.SemaphoreType.DMA,
    ],
)


def _finish_body(outp_ref, denp_ref, bias_ref, out_ref):
    num = jnp.sum(outp_ref[...], axis=0)
    den = jnp.sum(denp_ref[...], axis=0) + 1e-16
    outT = num / den[None, :] + bias_ref[...][:, None]
    eye = jnp.asarray(
        lax.broadcasted_iota(jnp.int32, (D_OUT, D_OUT), 0)
        == lax.broadcasted_iota(jnp.int32, (D_OUT, D_OUT), 1),
        dtype=jnp.float32)
    out_ref[...] = lax.dot_general(outT, eye, (((0,), (0,)), ((), ())),
                                   preferred_element_type=jnp.float32)


_finish = pl.pallas_call(
    _finish_body,
    out_shape=jax.ShapeDtypeStruct((N_DST, D_OUT), jnp.float32),
)


def kernel(x, edge_index, res_n_id, W, att_src, att_dst, bias):
    ei = edge_index.astype(jnp.int32)
    src = ei[0]
    dst = ei[1]
    rn = res_n_id.astype(jnp.int32)
    rn2 = jnp.pad(rn, (0, RN_PAD - N_DST)).reshape(RN_PAD // RN_CH, RN_CH)
    zeros = jnp.zeros((N_DST,), jnp.float32)

    hT, asrc, xad = _prep(x, W, att_src, att_dst)
    outp, denp = _edge(hT, asrc, xad, rn2, src, dst, zeros)
    return _finish(outp, denp, bias)


# speedup vs baseline: 1.1692x; 1.1692x over previous
import functools

import jax
import jax.numpy as jnp
from jax import lax
from jax.experimental import pallas as pl
from jax.experimental.pallas import tpu as pltpu
from jax.experimental.pallas import tpu_sc as plsc

N_SRC = 50000
N_DST = 10000
E = 800000
D_IN = 64
D_OUT = 16
NEG_SLOPE = 0.2

NCORES = 2
NSUB = 16
LANES = 16
EDGE_CHUNK = 2000
NFEAT = 4
NRANGE = 8
RN_CH = 128
RN_PAD = 80 * 128
RN_PER = 5
RN_ROWS = 8


def _prep_body(x_ref, w_ref, as_ref, ad_ref, hT_ref, asrc_ref, xad_ref):
    w = w_ref[...]
    x1 = x_ref[0:N_DST, :]
    hT = lax.dot_general(w, x1, (((0,), (1,)), ((), ())),
                         preferred_element_type=jnp.float32)
    hT_ref[...] = hT
    asrc_ref[...] = jnp.dot(as_ref[...], hT)
    wd = jnp.dot(w, ad_ref[...])
    xad_ref[...] = jnp.dot(x_ref[...], wd)


_prep = pl.pallas_call(
    _prep_body,
    out_shape=[
        jax.ShapeDtypeStruct((D_OUT, N_DST), jnp.float32),
        jax.ShapeDtypeStruct((N_DST,), jnp.float32),
        jax.ShapeDtypeStruct((N_SRC,), jnp.float32),
    ],
)


def _edge_body(hT, asrc, xadH, rn2H, srcH, dstH, zeros,
               outp, denp, adsth,
               asrc_v, adst_v,
               hc0, hc1, hc2, hc3, oc0, oc1, oc2, oc3,
               den_v, rn2_v,
               s_v0, s_v1, d_v0, d_v1,
               sem_s0, sem_s1, sem_d0, sem_d1, sem_g):
    c = lax.axis_index("c")
    s = lax.axis_index("s")
    s_v = (s_v0, s_v1)
    d_v = (d_v0, d_v1)
    sem_s = (sem_s0, sem_s1)
    sem_d = (sem_d0, sem_d1)
    hcol = (hc0, hc1, hc2, hc3)
    ocol = (oc0, oc1, oc2, oc3)

    fpart = s % NFEAT
    r = (c * NSUB + s) // NFEAT
    base = r * (E // NRANGE)
    n_chunks = (E // NRANGE) // EDGE_CHUNK

    def _start(j, b):
        off = base + j * EDGE_CHUNK
        pltpu.async_copy(srcH.at[pl.ds(off, EDGE_CHUNK)], s_v[b], sem_s[b])
        pltpu.async_copy(dstH.at[pl.ds(off, EDGE_CHUNK)], d_v[b], sem_d[b])

    def _wait(j, b):
        off = base + j * EDGE_CHUNK
        pltpu.make_async_copy(
            srcH.at[pl.ds(off, EDGE_CHUNK)], s_v[b], sem_s[b]).wait()
        pltpu.make_async_copy(
            dstH.at[pl.ds(off, EDGE_CHUNK)], d_v[b], sem_d[b]).wait()

    _start(0, 0)
    _start(1, 1)

    pltpu.sync_copy(rn2H.at[pl.ds(s * RN_ROWS, RN_ROWS)], rn2_v)
    for k in range(RN_PER):
        pltpu.async_copy(xadH.at[rn2_v.at[k]],
                         adst_v.at[pl.ds(k * RN_CH, RN_CH)], sem_g)

    pltpu.sync_copy(asrc, asrc_v)
    for k in range(NFEAT):
        pltpu.sync_copy(hT.at[fpart * NFEAT + k], hcol[k])
        pltpu.sync_copy(zeros, ocol[k])
    pltpu.sync_copy(zeros, den_v)

    for k in range(RN_PER):
        pltpu.make_async_copy(xadH.at[rn2_v.at[k]],
                              adst_v.at[pl.ds(k * RN_CH, RN_CH)], sem_g).wait()
    pltpu.sync_copy(adst_v.at[pl.ds(0, RN_PER * RN_CH)],
                    adsth.at[c, pl.ds(s * RN_PER * RN_CH, RN_PER * RN_CH)])
    plsc.subcore_barrier()
    pltpu.sync_copy(adsth.at[c], adst_v)

    def chunk_pair(jo, _):
        for b in range(2):
            j = 2 * jo + b
            _wait(j, b)

            @plsc.parallel_loop(0, EDGE_CHUNK // LANES, unroll=8)
            def _(i):
                sl = pl.ds(i * LANES, LANES)
                si = s_v[b][sl]
                di = d_v[b][sl]
                a = (plsc.load_gather(asrc_v, [si])
                     + plsc.load_gather(adst_v, [di]))
                a = jnp.where(a >= 0.0, a, a * NEG_SLOPE)
                ex = jnp.exp(a)
                plsc.addupdate_scatter(den_v, [di], ex)
                for k in range(NFEAT):
                    hv = plsc.load_gather(hcol[k], [si])
                    plsc.addupdate_scatter(ocol[k], [di], ex * hv)

            @pl.when(j + 2 < n_chunks)
            def _():
                _start(j + 2, b)
        return 0
    lax.fori_loop(0, n_chunks // 2, chunk_pair, 0)

    for k in range(NFEAT):
        pltpu.sync_copy(ocol[k], outp.at[r, fpart * NFEAT + k])

    @pl.when(fpart == 0)
    def _():
        pltpu.sync_copy(den_v, denp.at[r])


_edge = pl.kernel(
    _edge_body,
    out_type=[
        jax.ShapeDtypeStruct((NRANGE, D_OUT, N_DST), jnp.float32),
        jax.ShapeDtypeStruct((NRANGE, N_DST), jnp.float32),
        jax.ShapeDtypeStruct((NCORES, RN_PAD), jnp.float32),
    ],
    mesh=plsc.VectorSubcoreMesh(core_axis_name="c", subcore_axis_name="s"),
    compiler_params=pltpu.CompilerParams(needs_layout_passes=False),
    scratch_types=[
        pltpu.VMEM((N_DST,), jnp.float32),
        pltpu.VMEM((RN_PAD,), jnp.float32),
        pltpu.VMEM((N_DST,), jnp.float32),
        pltpu.VMEM((N_DST,), jnp.float32),
        pltpu.VMEM((N_DST,), jnp.float32),
        pltpu.VMEM((N_DST,), jnp.float32),
        pltpu.VMEM((N_DST,), jnp.float32),
        pltpu.VMEM((N_DST,), jnp.float32),
        pltpu.VMEM((N_DST,), jnp.float32),
        pltpu.VMEM((N_DST,), jnp.float32),
        pltpu.VMEM((N_DST,), jnp.float32),
        pltpu.VMEM((RN_ROWS, RN_CH), jnp.int32),
        pltpu.VMEM((EDGE_CHUNK,), jnp.int32),
        pltpu.VMEM((EDGE_CHUNK,), jnp.int32),
        pltpu.VMEM((EDGE_CHUNK,), jnp.int32),
        pltpu.VMEM((EDGE_CHUNK,), jnp.int32),
        pltpu.SemaphoreType.DMA,
        pltpu.SemaphoreType.DMA,
        pltpu.SemaphoreType.DMA,
        pltpu.SemaphoreType.DMA,
        pltpu.SemaphoreType.DMA,
    ],
)


def _finish_body(outp_ref, denp_ref, bias_ref, out_ref):
    num = jnp.sum(outp_ref[...], axis=0)
    den = jnp.sum(denp_ref[...], axis=0) + 1e-16
    outT = num / den[None, :] + bias_ref[...][:, None]
    eye = jnp.asarray(
        lax.broadcasted_iota(jnp.int32, (D_OUT, D_OUT), 0)
        == lax.broadcasted_iota(jnp.int32, (D_OUT, D_OUT), 1),
        dtype=jnp.float32)
    out_ref[...] = lax.dot_general(outT, eye, (((0,), (0,)), ((), ())),
                                   preferred_element_type=jnp.float32)


_finish = pl.pallas_call(
    _finish_body,
    out_shape=jax.ShapeDtypeStruct((N_DST, D_OUT), jnp.float32),
)


def kernel(x, edge_index, res_n_id, W, att_src, att_dst, bias):
    ei = edge_index.astype(jnp.int32)
    src = ei[0]
    dst = ei[1]
    rn = res_n_id.astype(jnp.int32)
    chunks = jnp.pad(rn, (0, RN_PAD - N_DST)).reshape(RN_PAD // RN_CH, RN_CH)
    rows = jnp.arange(NSUB * RN_ROWS)
    src_row = jnp.clip((rows // RN_ROWS) * RN_PER + rows % RN_ROWS,
                       0, RN_PAD // RN_CH - 1)
    rn2 = jnp.where((rows % RN_ROWS < RN_PER)[:, None], chunks[src_row], 0)
    zeros = jnp.zeros((N_DST,), jnp.float32)

    hT, asrc, xad = _prep(x, W, att_src, att_dst)
    outp, denp, _ = _edge(hT, asrc, xad, rn2, src, dst, zeros)
    return _finish(outp, denp, bias)


# speedup vs baseline: 1.2406x; 1.0611x over previous
import functools

import jax
import jax.numpy as jnp
from jax import lax
from jax.experimental import pallas as pl
from jax.experimental.pallas import tpu as pltpu
from jax.experimental.pallas import tpu_sc as plsc

N_SRC = 50000
N_DST = 10000
E = 800000
D_IN = 64
D_OUT = 16
NEG_SLOPE = 0.2

NCORES = 2
NSUB = 16
LANES = 16
EDGE_CHUNK = 2000
NFEAT = 4
NRANGE = 8
RN_CH = 128
RN_PAD = 80 * 128
RN_PER = 5
RN_ROWS = 8


def _prep_body(x_ref, w_ref, as_ref, ad_ref, hT_ref, asrc_ref, xad_ref):
    w = w_ref[...]
    x1 = x_ref[0:N_DST, :]
    hT = lax.dot_general(w, x1, (((0,), (1,)), ((), ())),
                         preferred_element_type=jnp.float32)
    hT_ref[...] = hT
    asrc_ref[...] = jnp.dot(as_ref[...], hT)
    wd = jnp.dot(w, ad_ref[...])
    xad_ref[...] = jnp.dot(x_ref[...], wd)


_prep = pl.pallas_call(
    _prep_body,
    out_shape=[
        jax.ShapeDtypeStruct((D_OUT, N_DST), jnp.float32),
        jax.ShapeDtypeStruct((N_DST,), jnp.float32),
        jax.ShapeDtypeStruct((N_SRC,), jnp.float32),
    ],
)


def _edge_body(hT, asrc, xadH, rn2H, srcH, dstH,
               outp, denp, adsth,
               asrc_v, adst_v,
               hc0, hc1, hc2, hc3, oc0, oc1, oc2, oc3,
               den_v, rn2_v,
               s_v0, s_v1, d_v0, d_v1,
               sem_s0, sem_s1, sem_d0, sem_d1, sem_g, sem_t, sem_r):
    c = lax.axis_index("c")
    s = lax.axis_index("s")
    s_v = (s_v0, s_v1)
    d_v = (d_v0, d_v1)
    sem_s = (sem_s0, sem_s1)
    sem_d = (sem_d0, sem_d1)
    hcol = (hc0, hc1, hc2, hc3)
    ocol = (oc0, oc1, oc2, oc3)

    fpart = s % NFEAT
    r = (c * NSUB + s) // NFEAT
    base = r * (E // NRANGE)
    n_chunks = (E // NRANGE) // EDGE_CHUNK

    def _start(j, b):
        off = base + j * EDGE_CHUNK
        pltpu.async_copy(srcH.at[pl.ds(off, EDGE_CHUNK)], s_v[b], sem_s[b])
        pltpu.async_copy(dstH.at[pl.ds(off, EDGE_CHUNK)], d_v[b], sem_d[b])

    def _wait(j, b):
        off = base + j * EDGE_CHUNK
        pltpu.make_async_copy(
            srcH.at[pl.ds(off, EDGE_CHUNK)], s_v[b], sem_s[b]).wait()
        pltpu.make_async_copy(
            dstH.at[pl.ds(off, EDGE_CHUNK)], d_v[b], sem_d[b]).wait()

    _start(0, 0)
    _start(1, 1)

    rn2_src = rn2H.at[pl.ds(s * RN_ROWS, RN_ROWS)]
    pltpu.async_copy(rn2_src, rn2_v, sem_r)
    asrc_cp = pltpu.async_copy(asrc, asrc_v, sem_t)
    h_srcs = [hT.at[fpart * NFEAT + k] for k in range(NFEAT)]
    for k in range(NFEAT):
        pltpu.async_copy(h_srcs[k], hcol[k], sem_t)

    z = jnp.zeros((LANES,), jnp.float32)

    @plsc.parallel_loop(0, N_DST // LANES, unroll=4)
    def _(i):
        sl = pl.ds(i * LANES, LANES)
        den_v[sl] = z
        for k in range(NFEAT):
            ocol[k][sl] = z

    pltpu.make_async_copy(rn2_src, rn2_v, sem_r).wait()
    for k in range(RN_PER):
        pltpu.async_copy(xadH.at[rn2_v.at[k]],
                         adst_v.at[pl.ds(k * RN_CH, RN_CH)], sem_g)

    asrc_cp.wait()
    for k in range(NFEAT):
        pltpu.make_async_copy(h_srcs[k], hcol[k], sem_t).wait()
    for k in range(RN_PER):
        pltpu.make_async_copy(xadH.at[rn2_v.at[k]],
                              adst_v.at[pl.ds(k * RN_CH, RN_CH)], sem_g).wait()
    pltpu.sync_copy(adst_v.at[pl.ds(0, RN_PER * RN_CH)],
                    adsth.at[c, pl.ds(s * RN_PER * RN_CH, RN_PER * RN_CH)])
    plsc.subcore_barrier()
    pltpu.sync_copy(adsth.at[c], adst_v)

    def chunk_pair(jo, _):
        for b in range(2):
            j = 2 * jo + b
            _wait(j, b)

            @plsc.parallel_loop(0, EDGE_CHUNK // LANES, unroll=8)
            def _(i):
                sl = pl.ds(i * LANES, LANES)
                si = s_v[b][sl]
                di = d_v[b][sl]
                a = (plsc.load_gather(asrc_v, [si])
                     + plsc.load_gather(adst_v, [di]))
                a = jnp.where(a >= 0.0, a, a * NEG_SLOPE)
                ex = jnp.exp(a)
                plsc.addupdate_scatter(den_v, [di], ex)
                for k in range(NFEAT):
                    hv = plsc.load_gather(hcol[k], [si])
                    plsc.addupdate_scatter(ocol[k], [di], ex * hv)

            @pl.when(j + 2 < n_chunks)
            def _():
                _start(j + 2, b)
        return 0
    lax.fori_loop(0, n_chunks // 2, chunk_pair, 0)

    o_dsts = [outp.at[r, fpart * NFEAT + k] for k in range(NFEAT)]
    for k in range(NFEAT):
        pltpu.async_copy(ocol[k], o_dsts[k], sem_t)

    @pl.when(fpart == 0)
    def _():
        pltpu.async_copy(den_v, denp.at[r], sem_r)

    for k in range(NFEAT):
        pltpu.make_async_copy(ocol[k], o_dsts[k], sem_t).wait()

    @pl.when(fpart == 0)
    def _():
        pltpu.make_async_copy(den_v, denp.at[r], sem_r).wait()


_edge = pl.kernel(
    _edge_body,
    out_type=[
        jax.ShapeDtypeStruct((NRANGE, D_OUT, N_DST), jnp.float32),
        jax.ShapeDtypeStruct((NRANGE, N_DST), jnp.float32),
        jax.ShapeDtypeStruct((NCORES, RN_PAD), jnp.float32),
    ],
    mesh=plsc.VectorSubcoreMesh(core_axis_name="c", subcore_axis_name="s"),
    compiler_params=pltpu.CompilerParams(needs_layout_passes=False),
    scratch_types=[
        pltpu.VMEM((N_DST,), jnp.float32),
        pltpu.VMEM((RN_PAD,), jnp.float32),
        pltpu.VMEM((N_DST,), jnp.float32),
        pltpu.VMEM((N_DST,), jnp.float32),
        pltpu.VMEM((N_DST,), jnp.float32),
        pltpu.VMEM((N_DST,), jnp.float32),
        pltpu.VMEM((N_DST,), jnp.float32),
        pltpu.VMEM((N_DST,), jnp.float32),
        pltpu.VMEM((N_DST,), jnp.float32),
        pltpu.VMEM((N_DST,), jnp.float32),
        pltpu.VMEM((N_DST,), jnp.float32),
        pltpu.VMEM((RN_ROWS, RN_CH), jnp.int32),
        pltpu.VMEM((EDGE_CHUNK,), jnp.int32),
        pltpu.VMEM((EDGE_CHUNK,), jnp.int32),
        pltpu.VMEM((EDGE_CHUNK,), jnp.int32),
        pltpu.VMEM((EDGE_CHUNK,), jnp.int32),
        pltpu.SemaphoreType.DMA,
        pltpu.SemaphoreType.DMA,
        pltpu.SemaphoreType.DMA,
        pltpu.SemaphoreType.DMA,
        pltpu.SemaphoreType.DMA,
        pltpu.SemaphoreType.DMA,
        pltpu.SemaphoreType.DMA,
    ],
)


def _finish_body(outp_ref, denp_ref, bias_ref, out_ref):
    num = jnp.sum(outp_ref[...], axis=0)
    den = jnp.sum(denp_ref[...], axis=0) + 1e-16
    outT = num / den[None, :] + bias_ref[...][:, None]
    eye = jnp.asarray(
        lax.broadcasted_iota(jnp.int32, (D_OUT, D_OUT), 0)
        == lax.broadcasted_iota(jnp.int32, (D_OUT, D_OUT), 1),
        dtype=jnp.float32)
    out_ref[...] = lax.dot_general(outT, eye, (((0,), (0,)), ((), ())),
                                   preferred_element_type=jnp.float32)


_finish = pl.pallas_call(
    _finish_body,
    out_shape=jax.ShapeDtypeStruct((N_DST, D_OUT), jnp.float32),
)


def kernel(x, edge_index, res_n_id, W, att_src, att_dst, bias):
    ei = edge_index.astype(jnp.int32)
    src = ei[0]
    dst = ei[1]
    rn = res_n_id.astype(jnp.int32)
    chunks = jnp.pad(rn, (0, RN_PAD - N_DST)).reshape(RN_PAD // RN_CH, RN_CH)
    rows = jnp.arange(NSUB * RN_ROWS)
    src_row = jnp.clip((rows // RN_ROWS) * RN_PER + rows % RN_ROWS,
                       0, RN_PAD // RN_CH - 1)
    rn2 = jnp.where((rows % RN_ROWS < RN_PER)[:, None], chunks[src_row], 0)

    hT, asrc, xad = _prep(x, W, att_src, att_dst)
    outp, denp, _ = _edge(hT, asrc, xad, rn2, src, dst)
    return _finish(outp, denp, bias)


# speedup vs baseline: 1.2420x; 1.0011x over previous
import functools

import jax
import jax.numpy as jnp
from jax import lax
from jax.experimental import pallas as pl
from jax.experimental.pallas import tpu as pltpu
from jax.experimental.pallas import tpu_sc as plsc

N_SRC = 50000
N_DST = 10000
E = 800000
D_IN = 64
D_OUT = 16
NEG_SLOPE = 0.2

NCORES = 2
NSUB = 16
LANES = 16
EDGE_CHUNK = 2000
NFEAT = 4
NRANGE = 8
RN_CH = 128
RN_PAD = 80 * 128
RN_PER = 5
RN_ROWS = 8


def _prep_body(x_ref, w_ref, as_ref, ad_ref, hT_ref, asrc_ref, xad_ref):
    i = pl.program_id(0)
    w = w_ref[...]
    xblk = x_ref[...]
    wd = jnp.dot(w, ad_ref[...])
    xad_ref[0, 0, :] = jnp.dot(xblk, wd)

    @pl.when(i == 0)
    def _():
        hT = lax.dot_general(w, xblk, (((0,), (1,)), ((), ())),
                             preferred_element_type=jnp.float32)
        hT_ref[...] = hT
        asrc_ref[...] = jnp.dot(as_ref[...], hT)


_prep = pl.pallas_call(
    _prep_body,
    grid=(N_SRC // N_DST,),
    in_specs=[
        pl.BlockSpec((N_DST, D_IN), lambda i: (i, 0)),
        pl.BlockSpec((D_IN, D_OUT), lambda i: (0, 0)),
        pl.BlockSpec((D_OUT,), lambda i: (0,)),
        pl.BlockSpec((D_OUT,), lambda i: (0,)),
    ],
    out_specs=[
        pl.BlockSpec((D_OUT, N_DST), lambda i: (0, 0)),
        pl.BlockSpec((N_DST,), lambda i: (0,)),
        pl.BlockSpec((1, 1, N_DST), lambda i: (i, 0, 0)),
    ],
    out_shape=[
        jax.ShapeDtypeStruct((D_OUT, N_DST), jnp.float32),
        jax.ShapeDtypeStruct((N_DST,), jnp.float32),
        jax.ShapeDtypeStruct((N_SRC // N_DST, 1, N_DST), jnp.float32),
    ],
)


def _edge_body(hT, asrc, xadH, rn2H, srcH, dstH,
               outp, denp, adsth,
               asrc_v, adst_v,
               hc0, hc1, hc2, hc3, oc0, oc1, oc2, oc3,
               den_v, rn2_v,
               s_v0, s_v1, d_v0, d_v1,
               sem_s0, sem_s1, sem_d0, sem_d1, sem_g, sem_t, sem_r):
    c = lax.axis_index("c")
    s = lax.axis_index("s")
    s_v = (s_v0, s_v1)
    d_v = (d_v0, d_v1)
    sem_s = (sem_s0, sem_s1)
    sem_d = (sem_d0, sem_d1)
    hcol = (hc0, hc1, hc2, hc3)
    ocol = (oc0, oc1, oc2, oc3)

    fpart = s % NFEAT
    r = (c * NSUB + s) // NFEAT
    base = r * (E // NRANGE)
    n_chunks = (E // NRANGE) // EDGE_CHUNK

    def _start(j, b):
        off = base + j * EDGE_CHUNK
        pltpu.async_copy(srcH.at[pl.ds(off, EDGE_CHUNK)], s_v[b], sem_s[b])
        pltpu.async_copy(dstH.at[pl.ds(off, EDGE_CHUNK)], d_v[b], sem_d[b])

    def _wait(j, b):
        off = base + j * EDGE_CHUNK
        pltpu.make_async_copy(
            srcH.at[pl.ds(off, EDGE_CHUNK)], s_v[b], sem_s[b]).wait()
        pltpu.make_async_copy(
            dstH.at[pl.ds(off, EDGE_CHUNK)], d_v[b], sem_d[b]).wait()

    _start(0, 0)
    _start(1, 1)

    rn2_src = rn2H.at[pl.ds(s * RN_ROWS, RN_ROWS)]
    pltpu.async_copy(rn2_src, rn2_v, sem_r)
    asrc_cp = pltpu.async_copy(asrc, asrc_v, sem_t)
    h_srcs = [hT.at[fpart * NFEAT + k] for k in range(NFEAT)]
    for k in range(NFEAT):
        pltpu.async_copy(h_srcs[k], hcol[k], sem_t)

    z = jnp.zeros((LANES,), jnp.float32)

    @plsc.parallel_loop(0, N_DST // LANES, unroll=4)
    def _(i):
        sl = pl.ds(i * LANES, LANES)
        den_v[sl] = z
        for k in range(NFEAT):
            ocol[k][sl] = z

    pltpu.make_async_copy(rn2_src, rn2_v, sem_r).wait()
    for k in range(RN_PER):
        pltpu.async_copy(xadH.at[rn2_v.at[k]],
                         adst_v.at[pl.ds(k * RN_CH, RN_CH)], sem_g)

    asrc_cp.wait()
    for k in range(NFEAT):
        pltpu.make_async_copy(h_srcs[k], hcol[k], sem_t).wait()
    for k in range(RN_PER):
        pltpu.make_async_copy(xadH.at[rn2_v.at[k]],
                              adst_v.at[pl.ds(k * RN_CH, RN_CH)], sem_g).wait()
    pltpu.sync_copy(adst_v.at[pl.ds(0, RN_PER * RN_CH)],
                    adsth.at[c, pl.ds(s * RN_PER * RN_CH, RN_PER * RN_CH)])
    plsc.subcore_barrier()
    pltpu.sync_copy(adsth.at[c], adst_v)

    def chunk_pair(jo, _):
        for b in range(2):
            j = 2 * jo + b
            _wait(j, b)

            @plsc.parallel_loop(0, EDGE_CHUNK // LANES, unroll=8)
            def _(i):
                sl = pl.ds(i * LANES, LANES)
                si = s_v[b][sl]
                di = d_v[b][sl]
                a = (plsc.load_gather(asrc_v, [si])
                     + plsc.load_gather(adst_v, [di]))
                a = jnp.where(a >= 0.0, a, a * NEG_SLOPE)
                ex = jnp.exp(a)
                plsc.addupdate_scatter(den_v, [di], ex)
                for k in range(NFEAT):
                    hv = plsc.load_gather(hcol[k], [si])
                    plsc.addupdate_scatter(ocol[k], [di], ex * hv)

            @pl.when(j + 2 < n_chunks)
            def _():
                _start(j + 2, b)
        return 0
    lax.fori_loop(0, n_chunks // 2, chunk_pair, 0)

    o_dsts = [outp.at[r, fpart * NFEAT + k] for k in range(NFEAT)]
    for k in range(NFEAT):
        pltpu.async_copy(ocol[k], o_dsts[k], sem_t)

    @pl.when(fpart == 0)
    def _():
        pltpu.async_copy(den_v, denp.at[r], sem_r)

    for k in range(NFEAT):
        pltpu.make_async_copy(ocol[k], o_dsts[k], sem_t).wait()

    @pl.when(fpart == 0)
    def _():
        pltpu.make_async_copy(den_v, denp.at[r], sem_r).wait()


_edge = pl.kernel(
    _edge_body,
    out_type=[
        jax.ShapeDtypeStruct((NRANGE, D_OUT, N_DST), jnp.float32),
        jax.ShapeDtypeStruct((NRANGE, N_DST), jnp.float32),
        jax.ShapeDtypeStruct((NCORES, RN_PAD), jnp.float32),
    ],
    mesh=plsc.VectorSubcoreMesh(core_axis_name="c", subcore_axis_name="s"),
    compiler_params=pltpu.CompilerParams(needs_layout_passes=False),
    scratch_types=[
        pltpu.VMEM((N_DST,), jnp.float32),
        pltpu.VMEM((RN_PAD,), jnp.float32),
        pltpu.VMEM((N_DST,), jnp.float32),
        pltpu.VMEM((N_DST,), jnp.float32),
        pltpu.VMEM((N_DST,), jnp.float32),
        pltpu.VMEM((N_DST,), jnp.float32),
        pltpu.VMEM((N_DST,), jnp.float32),
        pltpu.VMEM((N_DST,), jnp.float32),
        pltpu.VMEM((N_DST,), jnp.float32),
        pltpu.VMEM((N_DST,), jnp.float32),
        pltpu.VMEM((N_DST,), jnp.float32),
        pltpu.VMEM((RN_ROWS, RN_CH), jnp.int32),
        pltpu.VMEM((EDGE_CHUNK,), jnp.int32),
        pltpu.VMEM((EDGE_CHUNK,), jnp.int32),
        pltpu.VMEM((EDGE_CHUNK,), jnp.int32),
        pltpu.VMEM((EDGE_CHUNK,), jnp.int32),
        pltpu.SemaphoreType.DMA,
        pltpu.SemaphoreType.DMA,
        pltpu.SemaphoreType.DMA,
        pltpu.SemaphoreType.DMA,
        pltpu.SemaphoreType.DMA,
        pltpu.SemaphoreType.DMA,
        pltpu.SemaphoreType.DMA,
    ],
)


def _finish_body(outp_ref, denp_ref, bias_ref, out_ref):
    num = jnp.sum(outp_ref[...], axis=0)
    den = jnp.sum(denp_ref[...], axis=0) + 1e-16
    outT = num / den[None, :] + bias_ref[...][:, None]
    eye = jnp.asarray(
        lax.broadcasted_iota(jnp.int32, (D_OUT, D_OUT), 0)
        == lax.broadcasted_iota(jnp.int32, (D_OUT, D_OUT), 1),
        dtype=jnp.float32)
    out_ref[...] = lax.dot_general(outT, eye, (((0,), (0,)), ((), ())),
                                   preferred_element_type=jnp.float32)


_finish = pl.pallas_call(
    _finish_body,
    out_shape=jax.ShapeDtypeStruct((N_DST, D_OUT), jnp.float32),
)


def kernel(x, edge_index, res_n_id, W, att_src, att_dst, bias):
    ei = edge_index.astype(jnp.int32)
    src = ei[0]
    dst = ei[1]
    rn = res_n_id.astype(jnp.int32)
    chunks = jnp.pad(rn, (0, RN_PAD - N_DST)).reshape(RN_PAD // RN_CH, RN_CH)
    rows = jnp.arange(NSUB * RN_ROWS)
    src_row = jnp.clip((rows // RN_ROWS) * RN_PER + rows % RN_ROWS,
                       0, RN_PAD // RN_CH - 1)
    rn2 = jnp.where((rows % RN_ROWS < RN_PER)[:, None], chunks[src_row], 0)

    hT, asrc, xad3 = _prep(x, W, att_src, att_dst)
    xad = xad3.reshape(N_SRC)
    outp, denp, _ = _edge(hT, asrc, xad, rn2, src, dst)
    return _finish(outp, denp, bias)
